# baseline (speedup 1.0000x reference)
import functools

import jax
import jax.numpy as jnp
from jax import lax
from jax.experimental import pallas as pl
from jax.experimental.pallas import tpu as pltpu
from jax.experimental.pallas import tpu_sc as plsc

VOCAB = 1000000
HIDDEN = 64
B = 4096
L = 200

NC = 2
NS = 16
NW = NC * NS
LANES = 16
NCOL = HIDDEN // LANES

BT = B * L
B_PER_W = BT // NW
CH = 256
SUB = 128
NSUB = CH // SUB
N_CHUNKS = B_PER_W // CH
TOKROWS = B_PER_W // SUB
SCALE = 8.0

_DNUM = lax.GatherDimensionNumbers(
    offset_dims=(), collapsed_slice_dims=(0,), start_index_map=(0,))


def _bcast(vec, j):
    return lax.gather(vec, jnp.full((LANES, 1), j, jnp.int32),
                      dimension_numbers=_DNUM, slice_sizes=(1,),
                      mode=lax.GatherScatterMode.PROMISE_IN_BOUNDS)


TWB = 2048
NTBLK = -(-VOCAB // (2 * TWB))
VROWS = NTBLK * TWB
VOCAB2 = 2 * VROWS


def _tt_body(in_ref, out_ref):
    x = in_ref[...]
    eye = jnp.eye(HIDDEN, dtype=jnp.float32)
    xt = lax.dot_general(x, eye, (((0,), (0,)), ((), ())),
                         preferred_element_type=jnp.float32)
    out_ref[...] = jnp.concatenate([xt[:TWB], xt[TWB:]], axis=1)


def _transpose_table(table_t):
    return pl.pallas_call(
        _tt_body,
        grid=(NTBLK,),
        in_specs=[pl.BlockSpec((HIDDEN, 2 * TWB), lambda i: (0, i))],
        out_specs=pl.BlockSpec((TWB, 2 * HIDDEN), lambda i: (i, 0)),
        out_shape=jax.ShapeDtypeStruct((VROWS, 2 * HIDDEN), jnp.float32),
    )(table_t)


BC = 128


def _ot_body(in_ref, out_ref):
    x = in_ref[...].reshape(BC, L // 2, 2 * HIDDEN)
    for lp in range(L // 2):
        st = x[:, lp, :].T
        out_ref[2 * lp] = st[:HIDDEN]
        out_ref[2 * lp + 1] = st[HIDDEN:]


def _transpose_out(out2):
    return pl.pallas_call(
        _ot_body,
        grid=(B // BC,),
        in_specs=[pl.BlockSpec((BC * L // 2, 2 * HIDDEN), lambda i: (i, 0))],
        out_specs=pl.BlockSpec((L, HIDDEN, BC), lambda i: (0, 0, i)),
        out_shape=jax.ShapeDtypeStruct((L, HIDDEN, B), jnp.float32),
        compiler_params=pltpu.CompilerParams(
            vmem_limit_bytes=100 * 1024 * 1024),
    )(out2)


def _body(tok_hbm, val_hbm, table_hbm, w_hbm, out_hbm,
          idx_all, val_all, rows_a, rows_b, out_a, out_b, w_v,
          semg_a, semg_b, semo_a, semo_b):
    wid = lax.axis_index("s") * NC + lax.axis_index("c")
    base_w = wid * B_PER_W

    pltpu.sync_copy(w_hbm, w_v)
    w8 = [w_v[pl.ds(c * LANES, LANES)] * SCALE for c in range(NCOL)]
    pltpu.sync_copy(tok_hbm.at[pl.ds(wid * TOKROWS, TOKROWS)], idx_all)
    pltpu.sync_copy(val_hbm.at[pl.ds(base_w, B_PER_W)], val_all)

    def remap_body(tr, carry):
        for q in range(SUB // LANES):
            t = idx_all[tr, pl.ds(q * LANES, LANES)]
            hi = lax.shift_left(lax.shift_right_logical(t, 12), 12)
            lo = lax.shift_left(lax.bitwise_and(t, TWB - 1), 1)
            s = lax.bitwise_and(lax.shift_right_logical(t, 11), 1)
            idx_all[tr, pl.ds(q * LANES, LANES)] = hi + lo + s
        return carry

    lax.fori_loop(0, TOKROWS, remap_body, 0)

    def fire_gather(ci, rows_x, semg_x):
        for k in range(NSUB):
            pltpu.async_copy(table_hbm.at[idx_all.at[ci * NSUB + k]],
                             rows_x.at[pl.ds(k * SUB, SUB)], semg_x)

    fire_gather(0, rows_a, semg_a)
    fire_gather(1, rows_b, semg_b)

    def slot_step(k, ci, rows_x, out_x, semg_x, semo_x):
        pltpu.make_async_copy(table_hbm.at[pl.ds(0, CH)], rows_x,
                              semg_x).wait()

        @pl.when(k > 0)
        def _():
            pltpu.make_async_copy(out_hbm.at[pl.ds(0, CH // 2)], out_x,
                                  semo_x).wait()

        def group_body(g, c2):
            vv = val_all[pl.ds(ci * CH + g * LANES, LANES)]
            for j in range(LANES):
                vb = _bcast(vv, j)
                r = g * LANES + j
                pr = g * 8 + j // 2
                pc = (j % 2) * HIDDEN
                for c in range(NCOL):
                    x = rows_x[r, pl.ds(c * LANES, LANES)]
                    out_x[pr, pl.ds(pc + c * LANES, LANES)] = (
                        x * SCALE + vb * w8[c])
            return c2

        lax.fori_loop(0, CH // LANES, group_body, 0)
        pltpu.async_copy(out_x, out_hbm.at[pl.ds((base_w + ci * CH) // 2,
                                                 CH // 2)], semo_x)

        @pl.when(ci + 2 < N_CHUNKS)
        def _():
            fire_gather(ci + 2, rows_x, semg_x)

    def pair_body(k, carry):
        slot_step(k, 2 * k, rows_a, out_a, semg_a, semo_a)
        slot_step(k, 2 * k + 1, rows_b, out_b, semg_b, semo_b)
        return carry

    lax.fori_loop(0, N_CHUNKS // 2, pair_body, 0)

    pltpu.make_async_copy(out_hbm.at[pl.ds(0, CH // 2)], out_a, semo_a).wait()
    pltpu.make_async_copy(out_hbm.at[pl.ds(0, CH // 2)], out_b, semo_b).wait()


@jax.jit
def _embed(tokens2d, values_flat, table_lin, w_flat):
    run = pl.kernel(
        _body,
        out_type=jax.ShapeDtypeStruct((BT // 2, 2 * HIDDEN), jnp.float32),
        mesh=plsc.VectorSubcoreMesh(core_axis_name="c", subcore_axis_name="s"),
        scratch_types=[
            pltpu.VMEM((TOKROWS, SUB), jnp.int32),
            pltpu.VMEM((B_PER_W,), jnp.float32),
            pltpu.VMEM((CH, HIDDEN), jnp.float32),
            pltpu.VMEM((CH, HIDDEN), jnp.float32),
            pltpu.VMEM((CH // 2, 2 * HIDDEN), jnp.float32),
            pltpu.VMEM((CH // 2, 2 * HIDDEN), jnp.float32),
            pltpu.VMEM((HIDDEN,), jnp.float32),
            pltpu.SemaphoreType.DMA, pltpu.SemaphoreType.DMA,
            pltpu.SemaphoreType.DMA, pltpu.SemaphoreType.DMA,
        ],
        compiler_params=pltpu.CompilerParams(use_tc_tiling_on_sc=False,
                                             needs_layout_passes=False),
    )
    return run(tokens2d, values_flat, table_lin, w_flat)


def kernel(tokens, values, table, W_val):
    tokens2d = tokens.reshape(BT // SUB, SUB)
    packed = _transpose_table(table.T)
    table_lin = packed.reshape(VOCAB2, HIDDEN)
    out2 = _embed(tokens2d, values.reshape(BT), table_lin,
                  W_val.reshape(HIDDEN))
    emb = jnp.transpose(_transpose_out(out2), (2, 0, 1))
    padding_mask = tokens != 0
    return emb, padding_mask


# speedup vs baseline: 1.1769x; 1.1769x over previous
import functools

import jax
import jax.numpy as jnp
from jax import lax
from jax.experimental import pallas as pl
from jax.experimental.pallas import tpu as pltpu
from jax.experimental.pallas import tpu_sc as plsc

VOCAB = 1000000
HIDDEN = 64
B = 4096
L = 200

NC = 2
NS = 16
NW = NC * NS
LANES = 16
NCOL = HIDDEN // LANES

BT = B * L
B_PER_W = BT // NW
CH = 256
SUB = 128
NSUB = CH // SUB
N_CHUNKS = B_PER_W // CH
TOKROWS = B_PER_W // SUB
SCALE = 8.0

_DNUM = lax.GatherDimensionNumbers(
    offset_dims=(), collapsed_slice_dims=(0,), start_index_map=(0,))


def _bcast(vec, j):
    return lax.gather(vec, jnp.full((LANES, 1), j, jnp.int32),
                      dimension_numbers=_DNUM, slice_sizes=(1,),
                      mode=lax.GatherScatterMode.PROMISE_IN_BOUNDS)


TWB = 8192
NTBLK = -(-VOCAB // (2 * TWB))
VROWS = NTBLK * TWB
VOCAB2 = 2 * VROWS


def _tt_body(in_ref, out_ref):
    x = in_ref[...]
    out_ref[...] = jnp.concatenate([x[:, :TWB].T, x[:, TWB:].T], axis=1)


def _transpose_table(table_t):
    return pl.pallas_call(
        _tt_body,
        grid=(NTBLK,),
        in_specs=[pl.BlockSpec((HIDDEN, 2 * TWB), lambda i: (0, i))],
        out_specs=pl.BlockSpec((TWB, 2 * HIDDEN), lambda i: (i, 0)),
        out_shape=jax.ShapeDtypeStruct((VROWS, 2 * HIDDEN), jnp.float32),
        compiler_params=pltpu.CompilerParams(
            vmem_limit_bytes=100 * 1024 * 1024),
    )(table_t)


BC = 128


def _ot_body(in_ref, out_ref):
    x = in_ref[...].reshape(BC, L // 2, 2 * HIDDEN)
    for lp in range(L // 2):
        st = x[:, lp, :].T
        out_ref[2 * lp] = st[:HIDDEN]
        out_ref[2 * lp + 1] = st[HIDDEN:]


def _transpose_out(out2):
    return pl.pallas_call(
        _ot_body,
        grid=(B // BC,),
        in_specs=[pl.BlockSpec((BC * L // 2, 2 * HIDDEN), lambda i: (i, 0))],
        out_specs=pl.BlockSpec((L, HIDDEN, BC), lambda i: (0, 0, i)),
        out_shape=jax.ShapeDtypeStruct((L, HIDDEN, B), jnp.float32),
        compiler_params=pltpu.CompilerParams(
            vmem_limit_bytes=100 * 1024 * 1024),
    )(out2)


def _body(tok_hbm, val_hbm, table_hbm, w_hbm, out_hbm,
          idx_all, val_all, rows_a, rows_b, out_a, out_b, w_v,
          semg_a, semg_b, semo_a, semo_b):
    wid = lax.axis_index("s") * NC + lax.axis_index("c")
    base_w = wid * B_PER_W

    pltpu.sync_copy(w_hbm, w_v)
    w8 = [w_v[pl.ds(c * LANES, LANES)] * SCALE for c in range(NCOL)]
    pltpu.sync_copy(tok_hbm.at[pl.ds(wid * TOKROWS, TOKROWS)], idx_all)
    pltpu.sync_copy(val_hbm.at[pl.ds(base_w, B_PER_W)], val_all)

    def remap_body(tr, carry):
        for q in range(SUB // LANES):
            t = idx_all[tr, pl.ds(q * LANES, LANES)]
            hi = lax.shift_left(lax.shift_right_logical(t, 14), 14)
            lo = lax.shift_left(lax.bitwise_and(t, TWB - 1), 1)
            s = lax.bitwise_and(lax.shift_right_logical(t, 13), 1)
            idx_all[tr, pl.ds(q * LANES, LANES)] = hi + lo + s
        return carry

    lax.fori_loop(0, TOKROWS, remap_body, 0)

    def fire_gather(ci, rows_x, semg_x):
        for k in range(NSUB):
            pltpu.async_copy(table_hbm.at[idx_all.at[ci * NSUB + k]],
                             rows_x.at[pl.ds(k * SUB, SUB)], semg_x)

    fire_gather(0, rows_a, semg_a)
    fire_gather(1, rows_b, semg_b)

    def slot_step(k, ci, rows_x, out_x, semg_x, semo_x):
        pltpu.make_async_copy(table_hbm.at[pl.ds(0, CH)], rows_x,
                              semg_x).wait()

        @pl.when(k > 0)
        def _():
            pltpu.make_async_copy(out_hbm.at[pl.ds(0, CH // 2)], out_x,
                                  semo_x).wait()

        def group_body(g, c2):
            vv = val_all[pl.ds(ci * CH + g * LANES, LANES)]
            for j in range(LANES):
                vb = _bcast(vv, j)
                r = g * LANES + j
                pr = g * 8 + j // 2
                pc = (j % 2) * HIDDEN
                for c in range(NCOL):
                    x = rows_x[r, pl.ds(c * LANES, LANES)]
                    out_x[pr, pl.ds(pc + c * LANES, LANES)] = (
                        x * SCALE + vb * w8[c])
            return c2

        lax.fori_loop(0, CH // LANES, group_body, 0)
        pltpu.async_copy(out_x, out_hbm.at[pl.ds((base_w + ci * CH) // 2,
                                                 CH // 2)], semo_x)

        @pl.when(ci + 2 < N_CHUNKS)
        def _():
            fire_gather(ci + 2, rows_x, semg_x)

    def pair_body(k, carry):
        slot_step(k, 2 * k, rows_a, out_a, semg_a, semo_a)
        slot_step(k, 2 * k + 1, rows_b, out_b, semg_b, semo_b)
        return carry

    lax.fori_loop(0, N_CHUNKS // 2, pair_body, 0)

    pltpu.make_async_copy(out_hbm.at[pl.ds(0, CH // 2)], out_a, semo_a).wait()
    pltpu.make_async_copy(out_hbm.at[pl.ds(0, CH // 2)], out_b, semo_b).wait()


@jax.jit
def _embed(tokens2d, values_flat, table_lin, w_flat):
    run = pl.kernel(
        _body,
        out_type=jax.ShapeDtypeStruct((BT // 2, 2 * HIDDEN), jnp.float32),
        mesh=plsc.VectorSubcoreMesh(core_axis_name="c", subcore_axis_name="s"),
        scratch_types=[
            pltpu.VMEM((TOKROWS, SUB), jnp.int32),
            pltpu.VMEM((B_PER_W,), jnp.float32),
            pltpu.VMEM((CH, HIDDEN), jnp.float32),
            pltpu.VMEM((CH, HIDDEN), jnp.float32),
            pltpu.VMEM((CH // 2, 2 * HIDDEN), jnp.float32),
            pltpu.VMEM((CH // 2, 2 * HIDDEN), jnp.float32),
            pltpu.VMEM((HIDDEN,), jnp.float32),
            pltpu.SemaphoreType.DMA, pltpu.SemaphoreType.DMA,
            pltpu.SemaphoreType.DMA, pltpu.SemaphoreType.DMA,
        ],
        compiler_params=pltpu.CompilerParams(use_tc_tiling_on_sc=False,
                                             needs_layout_passes=False),
    )
    return run(tokens2d, values_flat, table_lin, w_flat)


def kernel(tokens, values, table, W_val):
    tokens2d = tokens.reshape(BT // SUB, SUB)
    packed = _transpose_table(table.T)
    table_lin = packed.reshape(VOCAB2, HIDDEN)
    out2 = _embed(tokens2d, values.reshape(BT), table_lin,
                  W_val.reshape(HIDDEN))
    emb = jnp.transpose(_transpose_out(out2), (2, 0, 1))
    padding_mask = tokens != 0
    return emb, padding_mask


# speedup vs baseline: 1.2113x; 1.0292x over previous
import functools

import jax
import jax.numpy as jnp
from jax import lax
from jax.experimental import pallas as pl
from jax.experimental.pallas import tpu as pltpu
from jax.experimental.pallas import tpu_sc as plsc

VOCAB = 1000000
HIDDEN = 64
B = 4096
L = 200

NC = 2
NS = 16
NW = NC * NS
LANES = 16
NCOL = HIDDEN // LANES

BT = B * L
B_PER_W = BT // NW
CH = 256
SUB = 128
NSUB = CH // SUB
N_CHUNKS = B_PER_W // CH
TOKROWS = B_PER_W // SUB
SCALE = 8.0

_DNUM = lax.GatherDimensionNumbers(
    offset_dims=(), collapsed_slice_dims=(0,), start_index_map=(0,))


def _bcast(vec, j):
    return lax.gather(vec, jnp.full((LANES, 1), j, jnp.int32),
                      dimension_numbers=_DNUM, slice_sizes=(1,),
                      mode=lax.GatherScatterMode.PROMISE_IN_BOUNDS)


TWB = 16384
NTBLK = -(-VOCAB // (2 * TWB))
VROWS = NTBLK * TWB
VOCAB2 = 2 * VROWS


def _tt_body(in_ref, out_ref):
    x = in_ref[...]
    out_ref[...] = jnp.concatenate([x[:, :TWB].T, x[:, TWB:].T], axis=1)


def _transpose_table(table_t):
    return pl.pallas_call(
        _tt_body,
        grid=(NTBLK,),
        in_specs=[pl.BlockSpec((HIDDEN, 2 * TWB), lambda i: (0, i))],
        out_specs=pl.BlockSpec((TWB, 2 * HIDDEN), lambda i: (i, 0)),
        out_shape=jax.ShapeDtypeStruct((VROWS, 2 * HIDDEN), jnp.float32),
        compiler_params=pltpu.CompilerParams(
            vmem_limit_bytes=100 * 1024 * 1024),
    )(table_t)


BC = 256


def _ot_body(in_ref, out_ref):
    x = in_ref[...].reshape(BC, L // 2, 2 * HIDDEN)
    for lp in range(L // 2):
        st = x[:, lp, :].T
        out_ref[2 * lp] = st[:HIDDEN]
        out_ref[2 * lp + 1] = st[HIDDEN:]


def _transpose_out(out2):
    return pl.pallas_call(
        _ot_body,
        grid=(B // BC,),
        in_specs=[pl.BlockSpec((BC * L // 2, 2 * HIDDEN), lambda i: (i, 0))],
        out_specs=pl.BlockSpec((L, HIDDEN, BC), lambda i: (0, 0, i)),
        out_shape=jax.ShapeDtypeStruct((L, HIDDEN, B), jnp.float32),
        compiler_params=pltpu.CompilerParams(
            vmem_limit_bytes=100 * 1024 * 1024),
    )(out2)


def _body(tok_hbm, val_hbm, table_hbm, w_hbm, out_hbm,
          idx_all, val_all, rows_a, rows_b, out_a, out_b, w_v,
          semg_a, semg_b, semo_a, semo_b):
    wid = lax.axis_index("s") * NC + lax.axis_index("c")
    base_w = wid * B_PER_W

    pltpu.sync_copy(w_hbm, w_v)
    w8 = [w_v[pl.ds(c * LANES, LANES)] * SCALE for c in range(NCOL)]
    pltpu.sync_copy(tok_hbm.at[pl.ds(wid * TOKROWS, TOKROWS)], idx_all)
    pltpu.sync_copy(val_hbm.at[pl.ds(base_w, B_PER_W)], val_all)

    def remap_body(tr, carry):
        for q in range(SUB // LANES):
            t = idx_all[tr, pl.ds(q * LANES, LANES)]
            hi = lax.shift_left(lax.shift_right_logical(t, 15), 15)
            lo = lax.shift_left(lax.bitwise_and(t, TWB - 1), 1)
            s = lax.bitwise_and(lax.shift_right_logical(t, 14), 1)
            idx_all[tr, pl.ds(q * LANES, LANES)] = hi + lo + s
        return carry

    lax.fori_loop(0, TOKROWS, remap_body, 0)

    def fire_gather(ci, rows_x, semg_x):
        for k in range(NSUB):
            pltpu.async_copy(table_hbm.at[idx_all.at[ci * NSUB + k]],
                             rows_x.at[pl.ds(k * SUB, SUB)], semg_x)

    fire_gather(0, rows_a, semg_a)
    fire_gather(1, rows_b, semg_b)

    def slot_step(k, ci, rows_x, out_x, semg_x, semo_x):
        pltpu.make_async_copy(table_hbm.at[pl.ds(0, CH)], rows_x,
                              semg_x).wait()

        @pl.when(k > 0)
        def _():
            pltpu.make_async_copy(out_hbm.at[pl.ds(0, CH // 2)], out_x,
                                  semo_x).wait()

        def group_body(g, c2):
            vv = val_all[pl.ds(ci * CH + g * LANES, LANES)]
            for j in range(LANES):
                vb = _bcast(vv, j)
                r = g * LANES + j
                pr = g * 8 + j // 2
                pc = (j % 2) * HIDDEN
                for c in range(NCOL):
                    x = rows_x[r, pl.ds(c * LANES, LANES)]
                    out_x[pr, pl.ds(pc + c * LANES, LANES)] = (
                        x * SCALE + vb * w8[c])
            return c2

        lax.fori_loop(0, CH // LANES, group_body, 0)
        pltpu.async_copy(out_x, out_hbm.at[pl.ds((base_w + ci * CH) // 2,
                                                 CH // 2)], semo_x)

        @pl.when(ci + 2 < N_CHUNKS)
        def _():
            fire_gather(ci + 2, rows_x, semg_x)

    def pair_body(k, carry):
        slot_step(k, 2 * k, rows_a, out_a, semg_a, semo_a)
        slot_step(k, 2 * k + 1, rows_b, out_b, semg_b, semo_b)
        return carry

    lax.fori_loop(0, N_CHUNKS // 2, pair_body, 0)

    pltpu.make_async_copy(out_hbm.at[pl.ds(0, CH // 2)], out_a, semo_a).wait()
    pltpu.make_async_copy(out_hbm.at[pl.ds(0, CH // 2)], out_b, semo_b).wait()


@jax.jit
def _embed(tokens2d, values_flat, table_lin, w_flat):
    run = pl.kernel(
        _body,
        out_type=jax.ShapeDtypeStruct((BT // 2, 2 * HIDDEN), jnp.float32),
        mesh=plsc.VectorSubcoreMesh(core_axis_name="c", subcore_axis_name="s"),
        scratch_types=[
            pltpu.VMEM((TOKROWS, SUB), jnp.int32),
            pltpu.VMEM((B_PER_W,), jnp.float32),
            pltpu.VMEM((CH, HIDDEN), jnp.float32),
            pltpu.VMEM((CH, HIDDEN), jnp.float32),
            pltpu.VMEM((CH // 2, 2 * HIDDEN), jnp.float32),
            pltpu.VMEM((CH // 2, 2 * HIDDEN), jnp.float32),
            pltpu.VMEM((HIDDEN,), jnp.float32),
            pltpu.SemaphoreType.DMA, pltpu.SemaphoreType.DMA,
            pltpu.SemaphoreType.DMA, pltpu.SemaphoreType.DMA,
        ],
        compiler_params=pltpu.CompilerParams(use_tc_tiling_on_sc=False,
                                             needs_layout_passes=False),
    )
    return run(tokens2d, values_flat, table_lin, w_flat)


def kernel(tokens, values, table, W_val):
    tokens2d = tokens.reshape(BT // SUB, SUB)
    packed = _transpose_table(table.T)
    table_lin = packed.reshape(VOCAB2, HIDDEN)
    out2 = _embed(tokens2d, values.reshape(BT), table_lin,
                  W_val.reshape(HIDDEN))
    emb = jnp.transpose(_transpose_out(out2), (2, 0, 1))
    padding_mask = tokens != 0
    return emb, padding_mask


# speedup vs baseline: 1.2125x; 1.0010x over previous
import jax
import jax.numpy as jnp
from jax import lax
from jax.experimental import pallas as pl
from jax.experimental.pallas import tpu as pltpu
from jax.experimental.pallas import tpu_sc as plsc

VOCAB = 1000000
HIDDEN = 64
B = 4096
L = 200

NC = 2
NS = 16
NW = NC * NS
LANES = 16
NCOL = HIDDEN // LANES

BT = B * L
B_PER_W = BT // NW
CH = 256
SUB = 128
NSUB = CH // SUB
N_CHUNKS = B_PER_W // CH
TOKROWS = B_PER_W // SUB
SCALE = 8.0

_DNUM = lax.GatherDimensionNumbers(
    offset_dims=(), collapsed_slice_dims=(0,), start_index_map=(0,))


def _bcast(vec, j):
    return lax.gather(vec, jnp.full((LANES, 1), j, jnp.int32),
                      dimension_numbers=_DNUM, slice_sizes=(1,),
                      mode=lax.GatherScatterMode.PROMISE_IN_BOUNDS)


TWB = 16384
NTBLK = -(-VOCAB // (2 * TWB))
VROWS = NTBLK * TWB
VOCAB2 = 2 * VROWS


def _tt_body(in_ref, out_ref):
    x = in_ref[...]
    out_ref[...] = jnp.concatenate([x[:, :TWB].T, x[:, TWB:].T], axis=1)


def _transpose_table(table_t):
    return pl.pallas_call(
        _tt_body,
        grid=(NTBLK,),
        in_specs=[pl.BlockSpec((HIDDEN, 2 * TWB), lambda i: (0, i))],
        out_specs=pl.BlockSpec((TWB, 2 * HIDDEN), lambda i: (i, 0)),
        out_shape=jax.ShapeDtypeStruct((VROWS, 2 * HIDDEN), jnp.float32),
        compiler_params=pltpu.CompilerParams(
            vmem_limit_bytes=100 * 1024 * 1024),
    )(table_t)


BC = 256


def _ot_body(in_ref, out_ref):
    x = in_ref[...].reshape(BC, L // 2, 2 * HIDDEN)
    for lp in range(L // 2):
        st = x[:, lp, :].T
        out_ref[2 * lp] = st[:HIDDEN]
        out_ref[2 * lp + 1] = st[HIDDEN:]


def _transpose_out(out2):
    return pl.pallas_call(
        _ot_body,
        grid=(B // BC,),
        in_specs=[pl.BlockSpec((BC * L // 2, 2 * HIDDEN), lambda i: (i, 0))],
        out_specs=pl.BlockSpec((L, HIDDEN, BC), lambda i: (0, 0, i)),
        out_shape=jax.ShapeDtypeStruct((L, HIDDEN, B), jnp.float32),
        compiler_params=pltpu.CompilerParams(
            vmem_limit_bytes=100 * 1024 * 1024),
    )(out2)


def _body(tok_hbm, val_hbm, table_hbm, w_hbm, out_hbm,
          idx_all, val_all, rows_a, rows_b, out_a, out_b, w_v,
          semg_a, semg_b, semo_a, semo_b):
    wid = lax.axis_index("s") * NC + lax.axis_index("c")
    base_w = wid * B_PER_W

    pltpu.sync_copy(w_hbm, w_v)
    w8 = [w_v[pl.ds(c * LANES, LANES)] * SCALE for c in range(NCOL)]
    pltpu.sync_copy(tok_hbm.at[pl.ds(wid * TOKROWS, TOKROWS)], idx_all)
    pltpu.sync_copy(val_hbm.at[pl.ds(base_w, B_PER_W)], val_all)

    def remap_body(tr, carry):
        for q in range(SUB // LANES):
            t = idx_all[tr, pl.ds(q * LANES, LANES)]
            hi = lax.shift_left(lax.shift_right_logical(t, 15), 15)
            lo = lax.shift_left(lax.bitwise_and(t, TWB - 1), 1)
            s = lax.bitwise_and(lax.shift_right_logical(t, 14), 1)
            idx_all[tr, pl.ds(q * LANES, LANES)] = hi + lo + s
        return carry

    lax.fori_loop(0, TOKROWS, remap_body, 0)

    def fire_gather(ci, rows_x, semg_x):
        for k in range(NSUB):
            pltpu.async_copy(table_hbm.at[idx_all.at[ci * NSUB + k]],
                             rows_x.at[pl.ds(k * SUB, SUB)], semg_x)

    fire_gather(0, rows_a, semg_a)
    fire_gather(1, rows_b, semg_b)

    def slot_step(k, ci, rows_x, out_x, semg_x, semo_x):
        pltpu.make_async_copy(table_hbm.at[pl.ds(0, CH)], rows_x,
                              semg_x).wait()

        @pl.when(k > 0)
        def _():
            pltpu.make_async_copy(out_hbm.at[pl.ds(0, CH // 2)], out_x,
                                  semo_x).wait()

        def group_body(g, c2):
            vv = val_all[pl.ds(ci * CH + g * LANES, LANES)]
            for j in range(LANES):
                vb = _bcast(vv, j)
                r = g * LANES + j
                pr = g * 8 + j // 2
                pc = (j % 2) * HIDDEN
                for c in range(NCOL):
                    x = rows_x[r, pl.ds(c * LANES, LANES)]
                    out_x[pr, pl.ds(pc + c * LANES, LANES)] = (
                        x * SCALE + vb * w8[c])
            return c2

        lax.fori_loop(0, CH // LANES, group_body, 0)
        pltpu.async_copy(out_x, out_hbm.at[pl.ds((base_w + ci * CH) // 2,
                                                 CH // 2)], semo_x)

        @pl.when(ci + 2 < N_CHUNKS)
        def _():
            fire_gather(ci + 2, rows_x, semg_x)

    def pair_body(k, carry):
        slot_step(k, 2 * k, rows_a, out_a, semg_a, semo_a)
        slot_step(k, 2 * k + 1, rows_b, out_b, semg_b, semo_b)
        return carry

    lax.fori_loop(0, N_CHUNKS // 2, pair_body, 0)

    pltpu.make_async_copy(out_hbm.at[pl.ds(0, CH // 2)], out_a, semo_a).wait()
    pltpu.make_async_copy(out_hbm.at[pl.ds(0, CH // 2)], out_b, semo_b).wait()


@jax.jit
def _embed(tokens2d, values_flat, table_lin, w_flat):
    run = pl.kernel(
        _body,
        out_type=jax.ShapeDtypeStruct((BT // 2, 2 * HIDDEN), jnp.float32),
        mesh=plsc.VectorSubcoreMesh(core_axis_name="c", subcore_axis_name="s"),
        scratch_types=[
            pltpu.VMEM((TOKROWS, SUB), jnp.int32),
            pltpu.VMEM((B_PER_W,), jnp.float32),
            pltpu.VMEM((CH, HIDDEN), jnp.float32),
            pltpu.VMEM((CH, HIDDEN), jnp.float32),
            pltpu.VMEM((CH // 2, 2 * HIDDEN), jnp.float32),
            pltpu.VMEM((CH // 2, 2 * HIDDEN), jnp.float32),
            pltpu.VMEM((HIDDEN,), jnp.float32),
            pltpu.SemaphoreType.DMA, pltpu.SemaphoreType.DMA,
            pltpu.SemaphoreType.DMA, pltpu.SemaphoreType.DMA,
        ],
        compiler_params=pltpu.CompilerParams(use_tc_tiling_on_sc=False,
                                             needs_layout_passes=False),
    )
    return run(tokens2d, values_flat, table_lin, w_flat)


def kernel(tokens, values, table, W_val):
    tokens2d = tokens.reshape(BT // SUB, SUB)
    packed = _transpose_table(table.T)
    table_lin = packed.reshape(VOCAB2, HIDDEN)
    out2 = _embed(tokens2d, values.reshape(BT), table_lin,
                  W_val.reshape(HIDDEN))
    emb = jnp.transpose(_transpose_out(out2), (2, 0, 1))
    padding_mask = tokens != 0
    return emb, padding_mask
